# baseline (speedup 1.0000x reference)
import jax
import jax.numpy as jnp
from jax import lax
from jax.experimental import pallas as pl
from jax.experimental.pallas import tpu as pltpu
from jax.experimental.pallas import tpu_sc as plsc

ALPHA = 0.2
N = 10000
E = 320000
F = 128
NC = 2
NS = 16
NW = NC * NS
EP = 10240
EPAD = NW * EP
CHUNK = 80
NCHUNK = EP // CHUNK
C3 = 40
CPB = 16
B3 = EP // C3 // CPB
NBUF = 3


def _k1_body(x_ref, w_ref, attn_ref, h_ref, s1_ref, s2_ref, m_ref):
    h = jnp.dot(x_ref[...], w_ref[...], preferred_element_type=jnp.float32)
    h_ref[...] = h
    a1 = attn_ref[0:F, :]
    a2 = attn_ref[F:, :]
    s1 = jnp.dot(h, a1, preferred_element_type=jnp.float32)
    s2 = jnp.dot(h, a2, preferred_element_type=jnp.float32)
    s1_ref[...] = s1
    s2_ref[...] = s2
    vm = jnp.min(s1) + jnp.min(s2)
    m = -jnp.maximum(vm, ALPHA * vm)
    m_ref[...] = jnp.full((1, F), m, dtype=jnp.float32)


def _k1(x, W, attn):
    return pl.pallas_call(
        _k1_body,
        out_shape=[
            jax.ShapeDtypeStruct((N, F), jnp.float32),
            jax.ShapeDtypeStruct((N, 1), jnp.float32),
            jax.ShapeDtypeStruct((N, 1), jnp.float32),
            jax.ShapeDtypeStruct((1, F), jnp.float32),
        ],
    )(x, W, attn)


def _k2_body(src_h, dst_h, s1_h, s2_h, m_h, num_h, rs_h,
             src_v, dst_v, num_v, s1_v, s2_v, m_v, z_v, rs_sh):
    cid = lax.axis_index("c")
    sid = lax.axis_index("s")
    wid = sid * NC + cid

    pltpu.sync_copy(src_h.at[wid], src_v)
    pltpu.sync_copy(dst_h.at[wid], dst_v)
    pltpu.sync_copy(s1_h, s1_v)
    pltpu.sync_copy(s2_h, s2_v)
    pltpu.sync_copy(m_h, m_v)

    def zb(i, _):
        z_v[pl.ds(i * 16, 16)] = jnp.zeros((16,), jnp.float32)
        return 0
    lax.fori_loop(0, 39, zb, 0)
    pltpu.sync_copy(z_v, rs_sh.at[pl.ds(sid * 624, 624)])

    @pl.when(sid == 0)
    def _():
        pltpu.sync_copy(z_v.at[pl.ds(0, 16)], rs_sh.at[pl.ds(9984, 16)])

    plsc.subcore_barrier()

    mreg = m_v[...]

    lane = lax.broadcasted_iota(jnp.int32, (16,), 0)

    def chunk(j, _):
        for k in range(CHUNK // 16):
            s16 = src_v[j, pl.ds(k * 16, 16)]
            d16 = dst_v[j, pl.ds(k * 16, 16)]
            a = plsc.load_gather(s1_v, [s16])
            b = plsc.load_gather(s2_v, [d16])
            v = a + b
            e = -jnp.maximum(v, ALPHA * v)
            gi = wid * EP + j * CHUNK + k * 16 + lane
            num16 = jnp.where(gi < E, jnp.exp(e - mreg), 0.0)
            num_v[j, pl.ds(k * 16, 16)] = num16
        pltpu.sync_copy(num_v.at[j], rs_sh.at[src_v.at[j]], add=True)
        return 0

    lax.fori_loop(0, NCHUNK, chunk, 0)

    pltpu.sync_copy(num_v, num_h.at[wid])
    plsc.subcore_barrier()

    @pl.when(sid == 0)
    def _():
        pltpu.sync_copy(rs_sh, rs_h.at[cid])


def _k2(src3, dst3, s1, s2, m16):
    mesh = plsc.VectorSubcoreMesh(core_axis_name="c", subcore_axis_name="s")
    return pl.kernel(
        _k2_body,
        out_type=[
            jax.ShapeDtypeStruct((NW, NCHUNK, CHUNK), jnp.float32),
            jax.ShapeDtypeStruct((NC, N), jnp.float32),
        ],
        mesh=mesh,
        compiler_params=pltpu.CompilerParams(needs_layout_passes=False),
        scratch_types=[
            pltpu.VMEM((NCHUNK, CHUNK), jnp.int32),
            pltpu.VMEM((NCHUNK, CHUNK), jnp.int32),
            pltpu.VMEM((NCHUNK, CHUNK), jnp.float32),
            pltpu.VMEM((N,), jnp.float32),
            pltpu.VMEM((N,), jnp.float32),
            pltpu.VMEM((16,), jnp.float32),
            pltpu.VMEM((624,), jnp.float32),
            pltpu.VMEM_SHARED((N,), jnp.float32),
        ],
    )(src3, dst3, s1, s2, m16)


def _bcast_lane(vec16, lane):
    idx = jnp.full((16, 1), lane, jnp.int32)
    dnums = lax.GatherDimensionNumbers(
        offset_dims=(), collapsed_slice_dims=(0,), start_index_map=(0,))
    return lax.gather(vec16, idx, dnums, (1,),
                      mode=lax.GatherScatterMode.PROMISE_IN_BOUNDS)


def _k3_body(h_hbm, src_h, dst_h, num_h, dum_h, hp_h,
             src_v, dst_v, num_v, rows_v, z_v, hp_sh, gsem, ssem):
    cid = lax.axis_index("c")
    sid = lax.axis_index("s")
    wid = sid * NC + cid

    def drain_scatter(b):
        pltpu.make_async_copy(dum_h, rows_v.at[b], ssem.at[b]).wait()

    def wait_gather(b):
        pltpu.make_async_copy(dum_h, rows_v.at[b], gsem.at[b]).wait()

    def zb(t, _):
        z_v[t // 8, pl.ds((t % 8) * 16, 16)] = jnp.zeros((16,), jnp.float32)
        return 0
    lax.fori_loop(0, 16 * 8, zb, 0)

    def zcopy(q, _):
        pltpu.sync_copy(z_v, hp_sh.at[pl.ds(sid * 624 + q * 16, 16)])
        return 0
    lax.fori_loop(0, 39, zcopy, 0)

    @pl.when(sid == 0)
    def _():
        pltpu.sync_copy(z_v, hp_sh.at[pl.ds(9984, 16)])

    plsc.subcore_barrier()

    nch = EP // C3

    def scale_rows(b, rlo, lanes, n16):
        for l in range(lanes):
            r = rlo + l
            b16 = _bcast_lane(n16, 16 - lanes + l)
            for f in range(F // 16):
                rows_v[b, r, pl.ds(f * 16, 16)] = (
                    rows_v[b, r, pl.ds(f * 16, 16)] * b16)

    def chunk_iter(j, _):
        ob = j // CPB
        jj = j - ob * CPB
        b = j % NBUF
        bn = (j + 1) % NBUF

        @pl.when(jj == 0)
        def _():
            @pl.when(j > 0)
            def _():
                drain_scatter((j - 1) % NBUF)
                drain_scatter((j - 2) % NBUF)
            pltpu.sync_copy(src_h.at[wid, ob], src_v)
            pltpu.sync_copy(dst_h.at[wid, ob], dst_v)
            pltpu.sync_copy(num_h.at[wid, ob], num_v)
            pltpu.async_copy(
                h_hbm.at[dst_v.at[0]], rows_v.at[b], gsem.at[b])

        @pl.when(jj >= 2)
        def _():
            drain_scatter(bn)

        @pl.when(jj <= CPB - 2)
        def _():
            pltpu.async_copy(
                h_hbm.at[dst_v.at[jj + 1]], rows_v.at[bn], gsem.at[bn])

        wait_gather(b)
        scale_rows(b, 0, 16, num_v[jj, pl.ds(0, 16)])
        scale_rows(b, 16, 16, num_v[jj, pl.ds(16, 16)])
        scale_rows(b, 32, 8, num_v[jj, pl.ds(C3 - 16, 16)])
        pltpu.async_copy(
            rows_v.at[b], hp_sh.at[src_v.at[jj]], ssem.at[b], add=True)
        return 0

    lax.fori_loop(0, nch, chunk_iter, 0)
    drain_scatter((nch - 1) % NBUF)
    drain_scatter((nch - 2) % NBUF)
    plsc.subcore_barrier()

    pltpu.sync_copy(hp_sh.at[pl.ds(sid * 624, 624)],
                    hp_h.at[cid, pl.ds(sid * 624, 624)])

    @pl.when(sid == 0)
    def _():
        pltpu.sync_copy(hp_sh.at[pl.ds(9984, 16)],
                        hp_h.at[cid, pl.ds(9984, 16)])


def _k3(h, src4, dst4, num4, dum):
    mesh = plsc.VectorSubcoreMesh(core_axis_name="c", subcore_axis_name="s")
    return pl.kernel(
        _k3_body,
        out_type=jax.ShapeDtypeStruct((NC, N, F), jnp.float32),
        mesh=mesh,
        compiler_params=pltpu.CompilerParams(needs_layout_passes=False),
        scratch_types=[
            pltpu.VMEM((CPB, C3), jnp.int32),
            pltpu.VMEM((CPB, C3), jnp.int32),
            pltpu.VMEM((CPB, C3), jnp.float32),
            pltpu.VMEM((NBUF, C3, F), jnp.float32),
            pltpu.VMEM((16, F), jnp.float32),
            pltpu.VMEM_SHARED((N, F), jnp.float32),
            pltpu.SemaphoreType.DMA((NBUF,)),
            pltpu.SemaphoreType.DMA((NBUF,)),
        ],
    )(h, src4, dst4, num4, dum)


def _k4_body(hp_ref, rs_ref, o_ref):
    u = hp_ref[0] + hp_ref[1]
    rs = rs_ref[0] + rs_ref[1]
    recip = jnp.where(rs > 0, 1.0 / rs, 0.0)
    t = u * recip
    o_ref[...] = jnp.where(t > 0, t, jnp.exp(t) - 1.0)


def _k4(hp, rsp):
    grid = 10
    rows = N // grid
    return pl.pallas_call(
        _k4_body,
        grid=(grid,),
        in_specs=[
            pl.BlockSpec((NC, rows, F), lambda i: (0, i, 0)),
            pl.BlockSpec((NC, rows, 1), lambda i: (0, i, 0)),
        ],
        out_specs=pl.BlockSpec((rows, F), lambda i: (i, 0)),
        out_shape=jax.ShapeDtypeStruct((N, F), jnp.float32),
    )(hp, rsp)


@jax.jit
def kernel(input, edge, W, attn):
    pad = jnp.zeros((EPAD - E,), jnp.int32)
    src3 = jnp.concatenate([edge[0], pad]).reshape(NW, NCHUNK, CHUNK)
    dst3 = jnp.concatenate([edge[1], pad]).reshape(NW, NCHUNK, CHUNK)
    h, s1, s2, mrow = _k1(input, W, attn)
    num3, rs = _k2(src3, dst3, s1.reshape(N), s2.reshape(N), mrow[0, :16])
    blk = (NW, B3, CPB, C3)
    dum = jnp.zeros((C3, F), jnp.float32)
    hp = _k3(h, src3.reshape(blk), dst3.reshape(blk), num3.reshape(blk), dum)
    return _k4(hp, rs.reshape(NC, N, 1))


# speedup vs baseline: 1.9997x; 1.9997x over previous
import jax
import jax.numpy as jnp
from jax import lax
from jax.experimental import pallas as pl
from jax.experimental.pallas import tpu as pltpu
from jax.experimental.pallas import tpu_sc as plsc

ALPHA = 0.2
N = 10000
E = 320000
F = 128
NC = 2
NS = 16
NW = NC * NS
EP = 10240
EPAD = NW * EP
CHUNK = 80
NCHUNK = EP // CHUNK
C3 = 40
CPB = 16
B3 = EP // C3 // CPB
NBUF = 3


def _k1_body(x_ref, w_ref, attn_ref, h_ref, s1_ref, s2_ref, m_ref):
    h = jnp.dot(x_ref[...], w_ref[...], preferred_element_type=jnp.float32)
    h_ref[...] = h
    a1 = attn_ref[0:F, :]
    a2 = attn_ref[F:, :]
    s1 = jnp.dot(h, a1, preferred_element_type=jnp.float32)
    s2 = jnp.dot(h, a2, preferred_element_type=jnp.float32)
    s1_ref[...] = s1
    s2_ref[...] = s2
    vm = jnp.min(s1) + jnp.min(s2)
    m = -jnp.maximum(vm, ALPHA * vm)
    m_ref[...] = jnp.full((1, F), m, dtype=jnp.float32)


def _k1(x, W, attn):
    return pl.pallas_call(
        _k1_body,
        out_shape=[
            jax.ShapeDtypeStruct((N, F), jnp.float32),
            jax.ShapeDtypeStruct((N, 1), jnp.float32),
            jax.ShapeDtypeStruct((N, 1), jnp.float32),
            jax.ShapeDtypeStruct((1, F), jnp.float32),
        ],
    )(x, W, attn)


def _k2_body(src_h, dst_h, s1_h, s2_h, m_h, num_h, rs_h,
             src_v, dst_v, num_v, s1_v, s2_v, m_v, z_v, rs_sh):
    cid = lax.axis_index("c")
    sid = lax.axis_index("s")
    wid = sid * NC + cid

    pltpu.sync_copy(src_h.at[wid], src_v)
    pltpu.sync_copy(dst_h.at[wid], dst_v)
    pltpu.sync_copy(s1_h, s1_v)
    pltpu.sync_copy(s2_h, s2_v)
    pltpu.sync_copy(m_h, m_v)

    def zb(i, _):
        z_v[pl.ds(i * 16, 16)] = jnp.zeros((16,), jnp.float32)
        return 0
    lax.fori_loop(0, 39, zb, 0)
    pltpu.sync_copy(z_v, rs_sh.at[pl.ds(sid * 624, 624)])

    @pl.when(sid == 0)
    def _():
        pltpu.sync_copy(z_v.at[pl.ds(0, 16)], rs_sh.at[pl.ds(9984, 16)])

    plsc.subcore_barrier()

    mreg = m_v[...]

    lane = lax.broadcasted_iota(jnp.int32, (16,), 0)

    def chunk(j, _):
        for k in range(CHUNK // 16):
            s16 = src_v[j, pl.ds(k * 16, 16)]
            d16 = dst_v[j, pl.ds(k * 16, 16)]
            a = plsc.load_gather(s1_v, [s16])
            b = plsc.load_gather(s2_v, [d16])
            v = a + b
            e = -jnp.maximum(v, ALPHA * v)
            gi = wid * EP + j * CHUNK + k * 16 + lane
            num16 = jnp.where(gi < E, jnp.exp(e - mreg), 0.0)
            num_v[j, pl.ds(k * 16, 16)] = num16
        pltpu.sync_copy(num_v.at[j], rs_sh.at[src_v.at[j]], add=True)
        return 0

    lax.fori_loop(0, NCHUNK, chunk, 0)

    pltpu.sync_copy(num_v, num_h.at[wid])
    plsc.subcore_barrier()

    @pl.when(sid == 0)
    def _():
        pltpu.sync_copy(rs_sh, rs_h.at[cid])


def _k2(src3, dst3, s1, s2, m16):
    mesh = plsc.VectorSubcoreMesh(core_axis_name="c", subcore_axis_name="s")
    return pl.kernel(
        _k2_body,
        out_type=[
            jax.ShapeDtypeStruct((NW, NCHUNK, CHUNK), jnp.float32),
            jax.ShapeDtypeStruct((NC, N), jnp.float32),
        ],
        mesh=mesh,
        compiler_params=pltpu.CompilerParams(needs_layout_passes=False),
        scratch_types=[
            pltpu.VMEM((NCHUNK, CHUNK), jnp.int32),
            pltpu.VMEM((NCHUNK, CHUNK), jnp.int32),
            pltpu.VMEM((NCHUNK, CHUNK), jnp.float32),
            pltpu.VMEM((N,), jnp.float32),
            pltpu.VMEM((N,), jnp.float32),
            pltpu.VMEM((16,), jnp.float32),
            pltpu.VMEM((624,), jnp.float32),
            pltpu.VMEM_SHARED((N,), jnp.float32),
        ],
    )(src3, dst3, s1, s2, m16)


def _bcast_lane(vec16, lane):
    idx = jnp.full((16, 1), lane, jnp.int32)
    dnums = lax.GatherDimensionNumbers(
        offset_dims=(), collapsed_slice_dims=(0,), start_index_map=(0,))
    return lax.gather(vec16, idx, dnums, (1,),
                      mode=lax.GatherScatterMode.PROMISE_IN_BOUNDS)


def _k3_body(h_hbm, src_h, dst_h, num_h, dum_h, hp_h,
             src_v, dst_v, num_v, rows_v, z_v, hp_sh, gsem, ssem):
    cid = lax.axis_index("c")
    sid = lax.axis_index("s")
    wid = sid * NC + cid

    def drain_scatter(b):
        pltpu.make_async_copy(dum_h, rows_v.at[b], ssem.at[b]).wait()

    def wait_gather(b):
        pltpu.make_async_copy(dum_h, rows_v.at[b], gsem.at[b]).wait()

    def zb(t, _):
        z_v[t // 8, pl.ds((t % 8) * 16, 16)] = jnp.zeros((16,), jnp.float32)
        return 0
    lax.fori_loop(0, 16 * 8, zb, 0)

    def zcopy(q, _):
        pltpu.sync_copy(z_v, hp_sh.at[pl.ds(sid * 624 + q * 16, 16)])
        return 0
    lax.fori_loop(0, 39, zcopy, 0)

    @pl.when(sid == 0)
    def _():
        pltpu.sync_copy(z_v, hp_sh.at[pl.ds(9984, 16)])

    plsc.subcore_barrier()

    nch = EP // C3

    def scale_rows(b, rlo, lanes, n16):
        for l in range(lanes):
            r = rlo + l
            b16 = _bcast_lane(n16, 16 - lanes + l)
            for f in range(F // 16):
                rows_v[b, r, pl.ds(f * 16, 16)] = (
                    rows_v[b, r, pl.ds(f * 16, 16)] * b16)

    def chunk_iter(j, _):
        ob = j // CPB
        jj = j - ob * CPB
        b = j % NBUF
        bn = (j + 1) % NBUF

        @pl.when(jj == 0)
        def _():
            @pl.when(j > 0)
            def _():
                drain_scatter((j - 1) % NBUF)
                drain_scatter((j - 2) % NBUF)
            pltpu.sync_copy(src_h.at[wid, ob], src_v)
            pltpu.sync_copy(dst_h.at[wid, ob], dst_v)
            pltpu.sync_copy(num_h.at[wid, ob], num_v)
            pltpu.async_copy(
                h_hbm.at[dst_v.at[0]], rows_v.at[b], gsem.at[b])

        @pl.when(jj >= 2)
        def _():
            drain_scatter(bn)

        @pl.when(jj <= CPB - 2)
        def _():
            pltpu.async_copy(
                h_hbm.at[dst_v.at[jj + 1]], rows_v.at[bn], gsem.at[bn])

        wait_gather(b)
        scale_rows(b, 0, 16, num_v[jj, pl.ds(0, 16)])
        scale_rows(b, 16, 16, num_v[jj, pl.ds(16, 16)])
        scale_rows(b, 32, 8, num_v[jj, pl.ds(C3 - 16, 16)])
        pltpu.async_copy(
            rows_v.at[b], hp_sh.at[src_v.at[jj]], ssem.at[b], add=True)
        return 0

    lax.fori_loop(0, nch, chunk_iter, 0)
    drain_scatter((nch - 1) % NBUF)
    drain_scatter((nch - 2) % NBUF)
    plsc.subcore_barrier()

    pltpu.sync_copy(hp_sh.at[pl.ds(sid * 624, 624)],
                    hp_h.at[cid, pl.ds(sid * 624, 624)])

    @pl.when(sid == 0)
    def _():
        pltpu.sync_copy(hp_sh.at[pl.ds(9984, 16)],
                        hp_h.at[cid, pl.ds(9984, 16)])


def _k3(h, src4, dst4, num4, dum):
    mesh = plsc.VectorSubcoreMesh(core_axis_name="c", subcore_axis_name="s")
    return pl.kernel(
        _k3_body,
        out_type=jax.ShapeDtypeStruct((NC, N, F), jnp.float32),
        mesh=mesh,
        compiler_params=pltpu.CompilerParams(needs_layout_passes=False),
        scratch_types=[
            pltpu.VMEM((CPB, C3), jnp.int32),
            pltpu.VMEM((CPB, C3), jnp.int32),
            pltpu.VMEM((CPB, C3), jnp.float32),
            pltpu.VMEM((NBUF, C3, F), jnp.float32),
            pltpu.VMEM((16, F), jnp.float32),
            pltpu.VMEM_SHARED((N, F), jnp.float32),
            pltpu.SemaphoreType.DMA((NBUF,)),
            pltpu.SemaphoreType.DMA((NBUF,)),
        ],
    )(h, src4, dst4, num4, dum)


def _k4_body(hp_ref, rs_ref, o_ref):
    u = hp_ref[0] + hp_ref[1]
    rs = rs_ref[0] + rs_ref[1]
    recip = jnp.where(rs > 0, 1.0 / rs, 0.0)
    t = u * recip
    o_ref[...] = jnp.where(t > 0, t, jnp.exp(t) - 1.0)


def _k4(hp, rsp):
    grid = 10
    rows = N // grid
    return pl.pallas_call(
        _k4_body,
        grid=(grid,),
        in_specs=[
            pl.BlockSpec((NC, rows, F), lambda i: (0, i, 0)),
            pl.BlockSpec((NC, rows, 1), lambda i: (0, i, 0)),
        ],
        out_specs=pl.BlockSpec((rows, F), lambda i: (i, 0)),
        out_shape=jax.ShapeDtypeStruct((N, F), jnp.float32),
    )(hp, rsp)


@jax.jit
def kernel(input, edge, W, attn):
    pad = jnp.arange(EPAD - E, dtype=jnp.int32) % N
    src3 = jnp.concatenate([edge[0], pad]).reshape(NW, NCHUNK, CHUNK)
    dst3 = jnp.concatenate([edge[1], pad]).reshape(NW, NCHUNK, CHUNK)
    h, s1, s2, mrow = _k1(input, W, attn)
    num3, rs = _k2(src3, dst3, s1.reshape(N), s2.reshape(N), mrow[0, :16])
    blk = (NW, B3, CPB, C3)
    dum = jnp.zeros((C3, F), jnp.float32)
    hp = _k3(h, src3.reshape(blk), dst3.reshape(blk), num3.reshape(blk), dum)
    return _k4(hp, rs.reshape(NC, N, 1))


# speedup vs baseline: 2.2724x; 1.1363x over previous
import jax
import jax.numpy as jnp
from jax import lax
from jax.experimental import pallas as pl
from jax.experimental.pallas import tpu as pltpu
from jax.experimental.pallas import tpu_sc as plsc

ALPHA = 0.2
N = 10000
E = 320000
F = 128
NC = 2
NS = 16
NW = NC * NS
EP = 10240
EPAD = NW * EP
CHUNK = 80
NCHUNK = EP // CHUNK
C3 = 40
CPB = 16
B3 = EP // C3 // CPB
NBUF = 3


def _k1_body(x_ref, w_ref, attn_ref, h_ref, s1_ref, s2_ref, m_ref):
    h = jnp.dot(x_ref[...], w_ref[...], preferred_element_type=jnp.float32)
    h_ref[...] = h
    a1 = attn_ref[0:F, :]
    a2 = attn_ref[F:, :]
    s1 = jnp.dot(h, a1, preferred_element_type=jnp.float32)
    s2 = jnp.dot(h, a2, preferred_element_type=jnp.float32)
    s1_ref[...] = s1
    s2_ref[...] = s2
    vm = jnp.min(s1) + jnp.min(s2)
    m = -jnp.maximum(vm, ALPHA * vm)
    m_ref[...] = jnp.full((1, F), m, dtype=jnp.float32)


def _k1(x, W, attn):
    return pl.pallas_call(
        _k1_body,
        out_shape=[
            jax.ShapeDtypeStruct((N, F), jnp.float32),
            jax.ShapeDtypeStruct((N, 1), jnp.float32),
            jax.ShapeDtypeStruct((N, 1), jnp.float32),
            jax.ShapeDtypeStruct((1, F), jnp.float32),
        ],
    )(x, W, attn)


def _k2_body(src_h, dst_h, s1_h, s2_h, m_h, num_h, rs_h,
             src_v, dst_v, num_v, s1_v, s2_v, m_v, z_v, rs_sh):
    cid = lax.axis_index("c")
    sid = lax.axis_index("s")
    wid = sid * NC + cid

    pltpu.sync_copy(src_h.at[wid], src_v)
    pltpu.sync_copy(dst_h.at[wid], dst_v)
    pltpu.sync_copy(s1_h, s1_v)
    pltpu.sync_copy(s2_h, s2_v)
    pltpu.sync_copy(m_h, m_v)

    def zb(i, _):
        z_v[pl.ds(i * 16, 16)] = jnp.zeros((16,), jnp.float32)
        return 0
    lax.fori_loop(0, 39, zb, 0)
    pltpu.sync_copy(z_v, rs_sh.at[pl.ds(sid * 624, 624)])

    @pl.when(sid == 0)
    def _():
        pltpu.sync_copy(z_v.at[pl.ds(0, 16)], rs_sh.at[pl.ds(9984, 16)])

    plsc.subcore_barrier()

    mreg = m_v[...]

    lane = lax.broadcasted_iota(jnp.int32, (16,), 0)

    def chunk(j, _):
        for k in range(CHUNK // 16):
            s16 = src_v[j, pl.ds(k * 16, 16)]
            d16 = dst_v[j, pl.ds(k * 16, 16)]
            a = plsc.load_gather(s1_v, [s16])
            b = plsc.load_gather(s2_v, [d16])
            v = a + b
            e = -jnp.maximum(v, ALPHA * v)
            gi = wid * EP + j * CHUNK + k * 16 + lane
            num16 = jnp.where(gi < E, jnp.exp(e - mreg), 0.0)
            num_v[j, pl.ds(k * 16, 16)] = num16
        pltpu.sync_copy(num_v.at[j], rs_sh.at[src_v.at[j]], add=True)
        return 0

    lax.fori_loop(0, NCHUNK, chunk, 0)

    pltpu.sync_copy(num_v, num_h.at[wid])
    plsc.subcore_barrier()

    @pl.when(sid == 0)
    def _():
        pltpu.sync_copy(rs_sh, rs_h.at[cid])


def _k2(src3, dst3, s1, s2, m16):
    mesh = plsc.VectorSubcoreMesh(core_axis_name="c", subcore_axis_name="s")
    return pl.kernel(
        _k2_body,
        out_type=[
            jax.ShapeDtypeStruct((NW, NCHUNK, CHUNK), jnp.float32),
            jax.ShapeDtypeStruct((NC, N), jnp.float32),
        ],
        mesh=mesh,
        compiler_params=pltpu.CompilerParams(needs_layout_passes=False),
        scratch_types=[
            pltpu.VMEM((NCHUNK, CHUNK), jnp.int32),
            pltpu.VMEM((NCHUNK, CHUNK), jnp.int32),
            pltpu.VMEM((NCHUNK, CHUNK), jnp.float32),
            pltpu.VMEM((N,), jnp.float32),
            pltpu.VMEM((N,), jnp.float32),
            pltpu.VMEM((16,), jnp.float32),
            pltpu.VMEM((624,), jnp.float32),
            pltpu.VMEM_SHARED((N,), jnp.float32),
        ],
    )(src3, dst3, s1, s2, m16)


def _bcast_lane(vec16, lane):
    idx = jnp.full((16, 1), lane, jnp.int32)
    dnums = lax.GatherDimensionNumbers(
        offset_dims=(), collapsed_slice_dims=(0,), start_index_map=(0,))
    return lax.gather(vec16, idx, dnums, (1,),
                      mode=lax.GatherScatterMode.PROMISE_IN_BOUNDS)


def _k3_body(h_hbm, src_h, dst_h, num_h, dum_h, hp_h,
             src_v, dst_v, num_v, rows_v, z_v, hp_sh, gsem, ssem, msem):
    cid = lax.axis_index("c")
    sid = lax.axis_index("s")
    wid = sid * NC + cid

    def drain_scatter(b):
        pltpu.make_async_copy(dum_h, rows_v.at[b], ssem.at[b]).wait()

    def wait_gather(b):
        pltpu.make_async_copy(dum_h, rows_v.at[b], gsem.at[b]).wait()

    def zb(t, _):
        z_v[t // 8, pl.ds((t % 8) * 16, 16)] = jnp.zeros((16,), jnp.float32)
        return 0
    lax.fori_loop(0, 16 * 8, zb, 0)

    def zcopy(q, _):
        pltpu.sync_copy(z_v, hp_sh.at[pl.ds(sid * 624 + q * 16, 16)])
        return 0
    lax.fori_loop(0, 39, zcopy, 0)

    @pl.when(sid == 0)
    def _():
        pltpu.sync_copy(z_v, hp_sh.at[pl.ds(9984, 16)])

    plsc.subcore_barrier()

    nch = EP // C3

    def scale_rows(b, rlo, lanes, n16):
        for l in range(lanes):
            r = rlo + l
            b16 = _bcast_lane(n16, 16 - lanes + l)
            for f in range(F // 16):
                rows_v[b, r, pl.ds(f * 16, 16)] = (
                    rows_v[b, r, pl.ds(f * 16, 16)] * b16)

    def chunk_iter(j, _):
        ob = j // CPB
        jj = j - ob * CPB
        mlo = (ob % 2) * CPB
        mlo2 = ((ob + 1) % 2) * CPB
        row = mlo + jj
        b = j % NBUF
        bn = (j + 1) % NBUF

        @pl.when(j == 0)
        def _():
            pltpu.sync_copy(src_h.at[wid, 0], src_v.at[pl.ds(0, CPB)])
            pltpu.sync_copy(dst_h.at[wid, 0], dst_v.at[pl.ds(0, CPB)])
            pltpu.sync_copy(num_h.at[wid, 0], num_v.at[pl.ds(0, CPB)])
            pltpu.async_copy(
                h_hbm.at[dst_v.at[0]], rows_v.at[b], gsem.at[b])

        @pl.when(j >= 2)
        def _():
            drain_scatter(bn)

        @pl.when(jnp.logical_and(jj == 2, ob < B3 - 1))
        def _():
            pltpu.async_copy(
                src_h.at[wid, ob + 1], src_v.at[pl.ds(mlo2, CPB)], msem)
            pltpu.async_copy(
                dst_h.at[wid, ob + 1], dst_v.at[pl.ds(mlo2, CPB)], msem)
            pltpu.async_copy(
                num_h.at[wid, ob + 1], num_v.at[pl.ds(mlo2, CPB)], msem)

        @pl.when(jj <= CPB - 2)
        def _():
            pltpu.async_copy(
                h_hbm.at[dst_v.at[row + 1]], rows_v.at[bn], gsem.at[bn])

        @pl.when(jnp.logical_and(jj == CPB - 1, j < nch - 1))
        def _():
            pltpu.make_async_copy(
                src_h.at[wid, 0], src_v.at[pl.ds(mlo2, CPB)], msem).wait()
            pltpu.make_async_copy(
                dst_h.at[wid, 0], dst_v.at[pl.ds(mlo2, CPB)], msem).wait()
            pltpu.make_async_copy(
                num_h.at[wid, 0], num_v.at[pl.ds(mlo2, CPB)], msem).wait()
            pltpu.async_copy(
                h_hbm.at[dst_v.at[mlo2]], rows_v.at[bn], gsem.at[bn])

        wait_gather(b)
        scale_rows(b, 0, 16, num_v[row, pl.ds(0, 16)])
        scale_rows(b, 16, 16, num_v[row, pl.ds(16, 16)])
        scale_rows(b, 32, 8, num_v[row, pl.ds(C3 - 16, 16)])
        pltpu.async_copy(
            rows_v.at[b], hp_sh.at[src_v.at[row]], ssem.at[b], add=True)
        return 0

    lax.fori_loop(0, nch, chunk_iter, 0)
    drain_scatter((nch - 1) % NBUF)
    drain_scatter((nch - 2) % NBUF)
    plsc.subcore_barrier()

    pltpu.sync_copy(hp_sh.at[pl.ds(sid * 624, 624)],
                    hp_h.at[cid, pl.ds(sid * 624, 624)])

    @pl.when(sid == 0)
    def _():
        pltpu.sync_copy(hp_sh.at[pl.ds(9984, 16)],
                        hp_h.at[cid, pl.ds(9984, 16)])


def _k3(h, src4, dst4, num4, dum):
    mesh = plsc.VectorSubcoreMesh(core_axis_name="c", subcore_axis_name="s")
    return pl.kernel(
        _k3_body,
        out_type=jax.ShapeDtypeStruct((NC, N, F), jnp.float32),
        mesh=mesh,
        compiler_params=pltpu.CompilerParams(needs_layout_passes=False),
        scratch_types=[
            pltpu.VMEM((2 * CPB, C3), jnp.int32),
            pltpu.VMEM((2 * CPB, C3), jnp.int32),
            pltpu.VMEM((2 * CPB, C3), jnp.float32),
            pltpu.VMEM((NBUF, C3, F), jnp.float32),
            pltpu.VMEM((16, F), jnp.float32),
            pltpu.VMEM_SHARED((N, F), jnp.float32),
            pltpu.SemaphoreType.DMA((NBUF,)),
            pltpu.SemaphoreType.DMA((NBUF,)),
            pltpu.SemaphoreType.DMA,
        ],
    )(h, src4, dst4, num4, dum)


def _k4_body(hp_ref, rs_ref, o_ref):
    u = hp_ref[0] + hp_ref[1]
    rs = rs_ref[0] + rs_ref[1]
    recip = jnp.where(rs > 0, 1.0 / rs, 0.0)
    t = u * recip
    o_ref[...] = jnp.where(t > 0, t, jnp.exp(t) - 1.0)


def _k4(hp, rsp):
    grid = 10
    rows = N // grid
    return pl.pallas_call(
        _k4_body,
        grid=(grid,),
        in_specs=[
            pl.BlockSpec((NC, rows, F), lambda i: (0, i, 0)),
            pl.BlockSpec((NC, rows, 1), lambda i: (0, i, 0)),
        ],
        out_specs=pl.BlockSpec((rows, F), lambda i: (i, 0)),
        out_shape=jax.ShapeDtypeStruct((N, F), jnp.float32),
    )(hp, rsp)


@jax.jit
def kernel(input, edge, W, attn):
    pad = jnp.arange(EPAD - E, dtype=jnp.int32) % N
    src3 = jnp.concatenate([edge[0], pad]).reshape(NW, NCHUNK, CHUNK)
    dst3 = jnp.concatenate([edge[1], pad]).reshape(NW, NCHUNK, CHUNK)
    h, s1, s2, mrow = _k1(input, W, attn)
    num3, rs = _k2(src3, dst3, s1.reshape(N), s2.reshape(N), mrow[0, :16])
    blk = (NW, B3, CPB, C3)
    dum = jnp.zeros((C3, F), jnp.float32)
    hp = _k3(h, src3.reshape(blk), dst3.reshape(blk), num3.reshape(blk), dum)
    return _k4(hp, rs.reshape(NC, N, 1))
